# baseline (speedup 1.0000x reference)
import functools

import jax
import jax.numpy as jnp
from jax import lax
from jax.experimental import pallas as pl
from jax.experimental.pallas import tpu as pltpu
from jax.experimental.pallas import tpu_sc as plsc

NE = 8192
ED = 32
BETA = 0.25
ROWS = 8 * 576
RB = 128
NRB = ROWS // RB
CCH = 2048
NCCH = NE // CCH
OHC = 1024
NOHC = NE // OHC
NC, NS, L = 2, 16, 16
NW = NC * NS
RPW = ROWS // NW
JW = RPW // L


GW = 128
GPC = CCH // GW


def _argmin_onehot_body(z_ref, e2_ref, idx_ref, oh_ref, se_ref, ids_ref):
    c = pl.program_id(1)

    @pl.when(c == 0)
    def _():
        @pl.when(pl.program_id(0) == 0)
        def _():
            e2 = e2_ref[...]
            se_ref[...] = (jnp.sum(e2 * e2, axis=1) * 0.25).reshape(1, NE)

        zb = z_ref[...]
        s_z = jnp.sum(zb * zb, axis=1, keepdims=True)

        def chunk(ci, carry):
            m, gi = carry
            mm2 = lax.dot_general(
                zb, e2_ref[pl.ds(ci * CCH, CCH), :], (((1,), (1,)), ((), ())),
                preferred_element_type=jnp.float32)
            for g in range(GPC):
                col = ci * GPC + g
                se_g = se_ref[0:1, pl.ds(col * GW, GW)]
                d = (s_z + se_g) - mm2[:, g * GW:(g + 1) * GW]
                upd = d < m
                m = jnp.where(upd, d, m)
                gi = jnp.where(upd, jnp.zeros((RB, GW), jnp.int32) + col, gi)
            return m, gi

        m0 = jnp.full((RB, GW), jnp.inf, jnp.float32)
        g0 = jnp.zeros((RB, GW), jnp.int32)
        m, gi = lax.fori_loop(0, NCCH, chunk, (m0, g0))
        cm = jnp.min(m, axis=1, keepdims=True)
        lane = lax.broadcasted_iota(jnp.int32, (RB, GW), 1)
        cand = gi * GW + lane
        bi = jnp.min(jnp.where(m == cm, cand, NE), axis=1, keepdims=True)
        ids_ref[...] = bi
        idx_ref[...] = bi.reshape(1, 1, RB)

    @pl.when(c == 1)
    def _():
        bi = ids_ref[...]
        lo = bi & (GW - 1)
        hi = bi >> 7
        liota = lax.broadcasted_iota(jnp.int32, (RB, GW), 1)
        bmat = jnp.where(liota == lo, 1.0, 0.0).astype(jnp.float32)
        giota = lax.broadcasted_iota(jnp.int32, (RB, NE // GW), 1)
        hmat = jnp.where(giota == hi, 1.0, 0.0).astype(jnp.float32)
        for g in range(NE // GW):
            oh_ref[:, g * GW:(g + 1) * GW] = bmat * hmat[:, g:g + 1]


def _scalar_body(cnt_ref, ps_ref, loss_ref, perp_ref):
    counts = cnt_ref[0:1, :] + cnt_ref[1:2, :]
    e = counts / float(ROWS)
    h = jnp.sum(e * jnp.log(e + 1e-10), axis=1, keepdims=True)
    perp_ref[...] = jnp.exp(-h)
    total = jnp.sum(ps_ref[...], keepdims=True)
    m = total / float(ROWS * ED)
    loss_ref[...] = m + BETA * m


def _sc_body(idx_hbm, emb_hbm, z_hbm, zeros_hbm,
               zq_hbm, counts_hbm, psum_hbm,
               idx_v, rows_v, z_v, st_v, ones_v, acc_v, hist_sh, sem):
    cid = lax.axis_index("c")
    sid = lax.axis_index("s")
    wid = sid * NC + cid
    base = wid * RPW

    pltpu.sync_copy(idx_hbm.at[wid], idx_v)
    pltpu.sync_copy(z_hbm.at[pl.ds(base, RPW)], z_v)
    ones_v[...] = jnp.full((L,), 1.0, jnp.float32)

    @pl.when(sid == 0)
    def _():
        pltpu.sync_copy(zeros_hbm, hist_sh)

    plsc.subcore_barrier()

    for j in range(JW):
        pltpu.async_copy(emb_hbm.at[idx_v.at[j]],
                         rows_v.at[pl.ds(j * L, L)], sem).wait()

    for j in range(JW):
        pltpu.sync_copy(ones_v, hist_sh.at[idx_v.at[j]], add=True)

    def body(r, acc):
        a0 = z_v[r, pl.ds(0, L)]
        b0 = rows_v[r, pl.ds(0, L)]
        d0 = b0 - a0
        st_v[r, pl.ds(0, L)] = a0 + d0
        a1 = z_v[r, pl.ds(L, L)]
        b1 = rows_v[r, pl.ds(L, L)]
        d1 = b1 - a1
        st_v[r, pl.ds(L, L)] = a1 + d1
        return acc + d0 * d0 + d1 * d1

    acc = lax.fori_loop(0, RPW, body, jnp.zeros((L,), jnp.float32))
    acc_v[...] = acc
    pltpu.sync_copy(acc_v, psum_hbm.at[pl.ds(wid * L, L)])
    pltpu.sync_copy(st_v, zq_hbm.at[pl.ds(base, RPW)])

    plsc.subcore_barrier()

    @pl.when(sid == 0)
    def _():
        pltpu.sync_copy(hist_sh, counts_hbm.at[pl.ds(cid * NE, NE)])


@functools.cache
def _sc_lookup_fn():
    mesh = plsc.VectorSubcoreMesh(core_axis_name="c", subcore_axis_name="s",
                                  num_cores=NC, num_subcores=NS)
    return pl.kernel(
        _sc_body,
        out_type=(
            jax.ShapeDtypeStruct((ROWS, ED), jnp.float32),
            jax.ShapeDtypeStruct((NC * NE,), jnp.float32),
            jax.ShapeDtypeStruct((NW * L,), jnp.float32),
        ),
        mesh=mesh,
        scratch_types=[
            pltpu.VMEM((JW, L), jnp.int32),
            pltpu.VMEM((RPW, ED), jnp.float32),
            pltpu.VMEM((RPW, ED), jnp.float32),
            pltpu.VMEM((RPW, ED), jnp.float32),
            pltpu.VMEM((L,), jnp.float32),
            pltpu.VMEM((L,), jnp.float32),
            pltpu.VMEM_SHARED((NE,), jnp.float32),
            pltpu.SemaphoreType.DMA,
        ],
        compiler_params=pltpu.CompilerParams(use_tc_tiling_on_sc=False),
    )


def kernel(z, emb_w):
    zp = jnp.transpose(z, (0, 2, 1))
    z_flat = zp.reshape(ROWS, ED)

    idx3, min_encodings = pl.pallas_call(
        _argmin_onehot_body,
        grid=(NRB, 2),
        in_specs=[
            pl.BlockSpec((RB, ED), lambda r, c: (r, 0)),
            pl.BlockSpec((NE, ED), lambda r, c: (0, 0)),
        ],
        out_specs=[
            pl.BlockSpec((1, 1, RB), lambda r, c: (r, 0, 0)),
            pl.BlockSpec((RB, NE), lambda r, c: (r, 0)),
        ],
        out_shape=[
            jax.ShapeDtypeStruct((NRB, 1, RB), jnp.int32),
            jax.ShapeDtypeStruct((ROWS, NE), jnp.float32),
        ],
        scratch_shapes=[
            pltpu.VMEM((1, NE), jnp.float32),
            pltpu.VMEM((RB, 1), jnp.int32),
        ],
    )(z_flat, emb_w * 2.0)

    idx3d = idx3.reshape(NW, JW, L)
    zq_flat, counts1, psum1 = _sc_lookup_fn()(
        idx3d, emb_w, z_flat, jnp.zeros((NE,), jnp.float32))

    loss2, perp2 = pl.pallas_call(
        _scalar_body,
        out_shape=(jax.ShapeDtypeStruct((1, 1), jnp.float32),
                   jax.ShapeDtypeStruct((1, 1), jnp.float32)),
    )(counts1.reshape(NC, NE), psum1.reshape(NW, L))

    loss = loss2[0, 0]
    perplexity = perp2[0, 0]
    z_q = jnp.transpose(zq_flat.reshape(8, 576, ED), (0, 2, 1))
    min_encoding_indices = idx3.reshape(ROWS, 1)
    return (loss, z_q, perplexity, min_encodings, min_encoding_indices)


# speedup vs baseline: 1.1981x; 1.1981x over previous
import functools

import jax
import jax.numpy as jnp
from jax import lax
from jax.experimental import pallas as pl
from jax.experimental.pallas import tpu as pltpu
from jax.experimental.pallas import tpu_sc as plsc

NE = 8192
ED = 32
BETA = 0.25
ROWS = 8 * 576
RB = 128
NRB = ROWS // RB
CCH = 2048
NCCH = NE // CCH
OHC = 1024
NOHC = NE // OHC
NC, NS, L = 2, 16, 16
NW = NC * NS
RPW = ROWS // NW
JW = RPW // L


GW = 128
GPC = CCH // GW


def _argmin_onehot_body(z_ref, emb_ref, idx_ref, oh_ref, e2_ref, se_ref):
    @pl.when(pl.program_id(0) == 0)
    def _():
        e2 = emb_ref[...] + emb_ref[...]
        e2_ref[...] = e2
        se_ref[...] = (jnp.sum(e2 * e2, axis=1) * 0.25).reshape(1, NE)

    zb = z_ref[...]
    s_z = jnp.sum(zb * zb, axis=1, keepdims=True)

    m = jnp.full((RB, GW), jnp.inf, jnp.float32)
    gi = jnp.zeros((RB, GW), jnp.int32)
    for ci in range(NCCH):
        mm2 = lax.dot_general(
            zb, e2_ref[ci * CCH:(ci + 1) * CCH, :], (((1,), (1,)), ((), ())),
            preferred_element_type=jnp.float32)
        for g in range(GPC):
            col = ci * GPC + g
            se_g = se_ref[0:1, col * GW:(col + 1) * GW]
            d = (s_z + se_g) - mm2[:, g * GW:(g + 1) * GW]
            upd = d < m
            m = jnp.where(upd, d, m)
            gi = jnp.where(upd, jnp.full((RB, GW), col, jnp.int32), gi)

    cm = jnp.min(m, axis=1, keepdims=True)
    lane = lax.broadcasted_iota(jnp.int32, (RB, GW), 1)
    cand = gi * GW + lane
    bi = jnp.min(jnp.where(m == cm, cand, NE), axis=1, keepdims=True)
    idx_ref[...] = bi.reshape(1, 1, RB)

    lo = bi & (GW - 1)
    hi = bi >> 7
    liota = lax.broadcasted_iota(jnp.int32, (RB, GW), 1)
    bmat = jnp.where(liota == lo, 1.0, 0.0).astype(jnp.float32)
    giota = lax.broadcasted_iota(jnp.int32, (RB, NE // GW), 1)
    hmat = jnp.where(giota == hi, 1.0, 0.0).astype(jnp.float32)
    for g in range(NE // GW):
        oh_ref[:, g * GW:(g + 1) * GW] = bmat * hmat[:, g:g + 1]


def _scalar_body(cnt_ref, ps_ref, loss_ref, perp_ref):
    counts = cnt_ref[0:1, :] + cnt_ref[1:2, :]
    e = counts / float(ROWS)
    h = jnp.sum(e * jnp.log(e + 1e-10), axis=1, keepdims=True)
    perp_ref[...] = jnp.exp(-h)
    total = jnp.sum(ps_ref[...], keepdims=True)
    m = total / float(ROWS * ED)
    loss_ref[...] = m + BETA * m


def _sc_body(idx_hbm, emb_hbm, z_hbm, zeros_hbm,
               zq_hbm, counts_hbm, psum_hbm,
               idx_v, rows_v, z_v, st_v, ones_v, acc_v, hist_sh, sem):
    cid = lax.axis_index("c")
    sid = lax.axis_index("s")
    wid = sid * NC + cid
    base = wid * RPW

    pltpu.sync_copy(idx_hbm.at[wid], idx_v)
    pltpu.sync_copy(z_hbm.at[pl.ds(base, RPW)], z_v)
    ones_v[...] = jnp.full((L,), 1.0, jnp.float32)

    @pl.when(sid == 0)
    def _():
        pltpu.sync_copy(zeros_hbm, hist_sh)

    plsc.subcore_barrier()

    for j in range(JW):
        pltpu.async_copy(emb_hbm.at[idx_v.at[j]],
                         rows_v.at[pl.ds(j * L, L)], sem).wait()

    for j in range(JW):
        pltpu.sync_copy(ones_v, hist_sh.at[idx_v.at[j]], add=True)

    def body(r, acc):
        a0 = z_v[r, pl.ds(0, L)]
        b0 = rows_v[r, pl.ds(0, L)]
        d0 = b0 - a0
        st_v[r, pl.ds(0, L)] = a0 + d0
        a1 = z_v[r, pl.ds(L, L)]
        b1 = rows_v[r, pl.ds(L, L)]
        d1 = b1 - a1
        st_v[r, pl.ds(L, L)] = a1 + d1
        return acc + d0 * d0 + d1 * d1

    acc = lax.fori_loop(0, RPW, body, jnp.zeros((L,), jnp.float32))
    acc_v[...] = acc
    pltpu.sync_copy(acc_v, psum_hbm.at[pl.ds(wid * L, L)])
    pltpu.sync_copy(st_v, zq_hbm.at[pl.ds(base, RPW)])

    plsc.subcore_barrier()

    @pl.when(sid == 0)
    def _():
        pltpu.sync_copy(hist_sh, counts_hbm.at[pl.ds(cid * NE, NE)])


@functools.cache
def _sc_lookup_fn():
    mesh = plsc.VectorSubcoreMesh(core_axis_name="c", subcore_axis_name="s",
                                  num_cores=NC, num_subcores=NS)
    return pl.kernel(
        _sc_body,
        out_type=(
            jax.ShapeDtypeStruct((ROWS, ED), jnp.float32),
            jax.ShapeDtypeStruct((NC * NE,), jnp.float32),
            jax.ShapeDtypeStruct((NW * L,), jnp.float32),
        ),
        mesh=mesh,
        scratch_types=[
            pltpu.VMEM((JW, L), jnp.int32),
            pltpu.VMEM((RPW, ED), jnp.float32),
            pltpu.VMEM((RPW, ED), jnp.float32),
            pltpu.VMEM((RPW, ED), jnp.float32),
            pltpu.VMEM((L,), jnp.float32),
            pltpu.VMEM((L,), jnp.float32),
            pltpu.VMEM_SHARED((NE,), jnp.float32),
            pltpu.SemaphoreType.DMA,
        ],
        compiler_params=pltpu.CompilerParams(use_tc_tiling_on_sc=False),
    )


def kernel(z, emb_w):
    zp = jnp.transpose(z, (0, 2, 1))
    z_flat = zp.reshape(ROWS, ED)

    idx3, min_encodings = pl.pallas_call(
        _argmin_onehot_body,
        grid=(NRB,),
        in_specs=[
            pl.BlockSpec((RB, ED), lambda r: (r, 0)),
            pl.BlockSpec((NE, ED), lambda r: (0, 0)),
        ],
        out_specs=[
            pl.BlockSpec((1, 1, RB), lambda r: (r, 0, 0)),
            pl.BlockSpec((RB, NE), lambda r: (r, 0)),
        ],
        out_shape=[
            jax.ShapeDtypeStruct((NRB, 1, RB), jnp.int32),
            jax.ShapeDtypeStruct((ROWS, NE), jnp.float32),
        ],
        scratch_shapes=[
            pltpu.VMEM((NE, ED), jnp.float32),
            pltpu.VMEM((1, NE), jnp.float32),
        ],
    )(z_flat, emb_w)

    idx3d = idx3.reshape(NW, JW, L)
    zq_flat, counts1, psum1 = _sc_lookup_fn()(
        idx3d, emb_w, z_flat, jnp.zeros((NE,), jnp.float32))

    loss2, perp2 = pl.pallas_call(
        _scalar_body,
        out_shape=(jax.ShapeDtypeStruct((1, 1), jnp.float32),
                   jax.ShapeDtypeStruct((1, 1), jnp.float32)),
    )(counts1.reshape(NC, NE), psum1.reshape(NW, L))

    loss = loss2[0, 0]
    perplexity = perp2[0, 0]
    z_q = jnp.transpose(zq_flat.reshape(8, 576, ED), (0, 2, 1))
    min_encoding_indices = idx3.reshape(ROWS, 1)
    return (loss, z_q, perplexity, min_encodings, min_encoding_indices)
